# baseline (speedup 1.0000x reference)
import functools

import jax
import jax.numpy as jnp
from jax import lax
from jax.experimental import pallas as pl
from jax.experimental.pallas import tpu as pltpu
from jax.experimental.pallas import tpu_sc as plsc

N_PRED = 1000000
EMB_DIM = 64
HID = 32
B = 16384
F = 26


PACK = 8
ND = PACK * HID
BLK = 8192
BLKR = BLK // PACK


def _mlp_body(x_ref, w1_ref, b1_ref, w2_ref, b2_ref, o_ref):
    x = x_ref[...]
    hT = lax.dot_general(w1_ref[...], x,
                         (((0,), (1,)), ((), ())),
                         preferred_element_type=jnp.float32)
    hT = jnp.maximum(hT + b1_ref[...], 0.0)
    lT = lax.dot_general(w2_ref[...], hT,
                         (((0,), (0,)), ((), ())),
                         preferred_element_type=jnp.float32)
    o_ref[...] = jax.nn.sigmoid(lT + b2_ref[...])[0]


def _prob_table(table, W1, b1, W2, b2, interpret=False):
    grid = pl.cdiv(N_PRED, BLK)
    return pl.pallas_call(
        _mlp_body,
        grid=(grid,),
        in_specs=[
            pl.BlockSpec((BLK, EMB_DIM), lambda i: (i, 0)),
            pl.BlockSpec((EMB_DIM, HID), lambda i: (0, 0)),
            pl.BlockSpec((HID, 1), lambda i: (0, 0)),
            pl.BlockSpec((HID, 1), lambda i: (0, 0)),
            pl.BlockSpec((1, 1), lambda i: (0, 0)),
        ],
        out_specs=pl.BlockSpec((BLK,), lambda i: (i,)),
        out_shape=jax.ShapeDtypeStruct((N_PRED,), jnp.float32),
        interpret=interpret,
    )(table, W1, b1.reshape(HID, 1), W2, b2.reshape(1, 1))


_NC, _NS = 2, 16
_NW = _NC * _NS
_TOT = B * F
_PER_W = _TOT // _NW


def _gather_body(ptab_hbm, idx_hbm, out_hbm, idx_v, val_v, sem):
    wid = lax.axis_index("s") * _NC + lax.axis_index("c")
    base = wid * _PER_W
    pltpu.sync_copy(idx_hbm.at[pl.ds(base, _PER_W)], idx_v)
    pltpu.async_copy(ptab_hbm.at[idx_v], val_v, sem).wait()
    pltpu.sync_copy(val_v, out_hbm.at[pl.ds(base, _PER_W)])


@functools.cache
def _make_gather():
    return pl.kernel(
        _gather_body,
        out_type=jax.ShapeDtypeStruct((_TOT,), jnp.float32),
        mesh=plsc.VectorSubcoreMesh(
            core_axis_name="c", subcore_axis_name="s",
            num_cores=_NC, num_subcores=_NS),
        scratch_types=[
            pltpu.VMEM((_PER_W,), jnp.int32),
            pltpu.VMEM((_PER_W,), jnp.float32),
            pltpu.SemaphoreType.DMA,
        ],
    )


_NBUF = 5
_CH = 8000
_NCH = N_PRED // _CH


def _dma_probe_body(x_hbm, o_ref, bufs, sems):
    for b in range(_NBUF):
        pltpu.make_async_copy(
            x_hbm.at[pl.ds(b * _CH, _CH), :], bufs.at[b], sems.at[b]).start()

    def outer(i, carry):
        for b in range(_NBUF):
            c = i * _NBUF + b
            pltpu.make_async_copy(
                x_hbm.at[pl.ds(0, _CH), :], bufs.at[b], sems.at[b]).wait()
            o_ref[...] = bufs[b, :8, :]

            @pl.when(c + _NBUF < _NCH)
            def _():
                pltpu.make_async_copy(
                    x_hbm.at[pl.ds((c + _NBUF) * _CH, _CH), :],
                    bufs.at[b], sems.at[b]).start()
        return carry

    lax.fori_loop(0, _NCH // _NBUF, outer, 0)


def _probe_body(x_ref, o_ref):
    o_ref[...] = x_ref[:8, :]


def kernel(predicate_indices, table, W1, b1, W2, b2):
    y = table.reshape(125000, 512)
    grid = pl.cdiv(125000, 4096)
    out = pl.pallas_call(
        _probe_body,
        grid=(grid,),
        in_specs=[pl.BlockSpec((4096, 512), lambda i: (i, 0))],
        out_specs=pl.BlockSpec((8, 512), lambda i: (i, 0)),
        out_shape=jax.ShapeDtypeStruct((31 * 8, 512), jnp.float32),
    )(y)
    return jnp.broadcast_to(out[0, 0], (B, F))


# speedup vs baseline: 1.1533x; 1.1533x over previous
import functools

import jax
import jax.numpy as jnp
from jax import lax
from jax.experimental import pallas as pl
from jax.experimental.pallas import tpu as pltpu
from jax.experimental.pallas import tpu_sc as plsc

N_PRED = 1000000
EMB_DIM = 64
HID = 32
B = 16384
F = 26


PACK = 8
ND = PACK * HID
BLK = 8192
BLKR = BLK // PACK


def _mlp_body(x_ref, w1_ref, b1_ref, w2_ref, b2_ref, o_ref):
    x = x_ref[...].astype(jnp.bfloat16)
    hT = lax.dot_general(w1_ref[...].astype(jnp.bfloat16), x,
                         (((0,), (1,)), ((), ())),
                         preferred_element_type=jnp.float32)
    hT = jnp.maximum(hT + b1_ref[...], 0.0).astype(jnp.bfloat16)
    lT = lax.dot_general(w2_ref[...].astype(jnp.bfloat16), hT,
                         (((0,), (0,)), ((), ())),
                         preferred_element_type=jnp.float32)
    o_ref[...] = jax.nn.sigmoid(lT + b2_ref[...])[0]


def _prob_table(table, W1, b1, W2, b2, interpret=False):
    grid = pl.cdiv(N_PRED, BLK)
    return pl.pallas_call(
        _mlp_body,
        grid=(grid,),
        in_specs=[
            pl.BlockSpec((BLK, EMB_DIM), lambda i: (i, 0)),
            pl.BlockSpec((EMB_DIM, HID), lambda i: (0, 0)),
            pl.BlockSpec((HID, 1), lambda i: (0, 0)),
            pl.BlockSpec((HID, 1), lambda i: (0, 0)),
            pl.BlockSpec((1, 1), lambda i: (0, 0)),
        ],
        out_specs=pl.BlockSpec((BLK,), lambda i: (i,)),
        out_shape=jax.ShapeDtypeStruct((N_PRED,), jnp.float32),
        interpret=interpret,
    )(table, W1, b1.reshape(HID, 1), W2, b2.reshape(1, 1))


_NC, _NS = 2, 16
_NW = _NC * _NS
_TOT = B * F
_PER_W = _TOT // _NW


def _gather_body(ptab_hbm, idx_hbm, out_hbm, idx_v, val_v, sem):
    wid = lax.axis_index("s") * _NC + lax.axis_index("c")
    base = wid * _PER_W
    pltpu.sync_copy(idx_hbm.at[pl.ds(base, _PER_W)], idx_v)
    pltpu.async_copy(ptab_hbm.at[idx_v], val_v, sem).wait()
    pltpu.sync_copy(val_v, out_hbm.at[pl.ds(base, _PER_W)])


@functools.cache
def _make_gather():
    return pl.kernel(
        _gather_body,
        out_type=jax.ShapeDtypeStruct((_TOT,), jnp.float32),
        mesh=plsc.VectorSubcoreMesh(
            core_axis_name="c", subcore_axis_name="s",
            num_cores=_NC, num_subcores=_NS),
        scratch_types=[
            pltpu.VMEM((_PER_W,), jnp.int32),
            pltpu.VMEM((_PER_W,), jnp.float32),
            pltpu.SemaphoreType.DMA,
        ],
    )


_GB = 832
_NBUF = 2
_NCHUNK = _PER_W // _GB


def _rowgather_body(table_hbm, idx_hbm, out_hbm, idx_v, buf, gsem, wsem):
    wid = lax.axis_index("s") * _NC + lax.axis_index("c")
    base = wid * _PER_W
    pltpu.sync_copy(idx_hbm.at[pl.ds(base, _PER_W)], idx_v)

    def _start_gather(c, b):
        pltpu.make_async_copy(
            table_hbm.at[idx_v.at[pl.ds(c * _GB, _GB)]],
            buf.at[b], gsem.at[b]).start()

    for b in range(_NBUF):
        _start_gather(b, b)

    def outer(i, carry):
        for b in range(_NBUF):
            c = i * _NBUF + b
            pltpu.make_async_copy(
                table_hbm.at[idx_v.at[pl.ds(0, _GB)]],
                buf.at[b], gsem.at[b]).wait()
            pltpu.make_async_copy(
                buf.at[b], out_hbm.at[pl.ds(base + c * _GB, _GB)],
                wsem.at[b]).start()

            @pl.when(c + _NBUF < _NCHUNK)
            def _():
                pltpu.make_async_copy(
                    buf.at[b], out_hbm.at[pl.ds(0, _GB)], wsem.at[b]).wait()
                _start_gather(c + _NBUF, b)
        return carry

    lax.fori_loop(0, _NCHUNK // _NBUF, outer, 0)
    for b in range(_NBUF):
        pltpu.make_async_copy(
            buf.at[b], out_hbm.at[pl.ds(0, _GB)], wsem.at[b]).wait()


@functools.cache
def _make_rowgather():
    return pl.kernel(
        _rowgather_body,
        out_type=jax.ShapeDtypeStruct((_TOT, EMB_DIM), jnp.float32),
        mesh=plsc.VectorSubcoreMesh(
            core_axis_name="c", subcore_axis_name="s",
            num_cores=_NC, num_subcores=_NS),
        scratch_types=[
            pltpu.VMEM((_PER_W,), jnp.int32),
            pltpu.VMEM((_NBUF, _GB, EMB_DIM), jnp.float32),
            pltpu.SemaphoreType.DMA((_NBUF,)),
            pltpu.SemaphoreType.DMA((_NBUF,)),
        ],
    )


def _mlp_rows(rows, W1, b1, W2, b2):
    grid = _TOT // BLK
    return pl.pallas_call(
        _mlp_body,
        grid=(grid,),
        in_specs=[
            pl.BlockSpec((BLK, EMB_DIM), lambda i: (i, 0)),
            pl.BlockSpec((EMB_DIM, HID), lambda i: (0, 0)),
            pl.BlockSpec((HID, 1), lambda i: (0, 0)),
            pl.BlockSpec((HID, 1), lambda i: (0, 0)),
            pl.BlockSpec((1, 1), lambda i: (0, 0)),
        ],
        out_specs=pl.BlockSpec((BLK,), lambda i: (i,)),
        out_shape=jax.ShapeDtypeStruct((_TOT,), jnp.float32),
    )(rows, W1, b1.reshape(HID, 1), W2, b2.reshape(1, 1))


def kernel(predicate_indices, table, W1, b1, W2, b2):
    ptab = _prob_table(table, W1, b1, W2, b2)
    flat_idx = predicate_indices.reshape(_TOT).astype(jnp.int32)
    probs = _make_gather()(ptab, flat_idx)
    return probs.reshape(B, F)


# speedup vs baseline: 3.8753x; 3.3602x over previous
import functools

import jax
import jax.numpy as jnp
from jax import lax
from jax.experimental import pallas as pl
from jax.experimental.pallas import tpu as pltpu
from jax.experimental.pallas import tpu_sc as plsc

N_PRED = 1000000
EMB_DIM = 64
HID = 32
B = 16384
F = 26


BLK = 8192


def _mlp_body(xt_ref, w1_ref, b1_ref, w2_ref, b2_ref, o_ref):
    xT = xt_ref[...].astype(jnp.bfloat16)
    hT = lax.dot_general(w1_ref[...].astype(jnp.bfloat16), xT,
                         (((0,), (0,)), ((), ())),
                         preferred_element_type=jnp.float32)
    hT = jnp.maximum(hT + b1_ref[...], 0.0).astype(jnp.bfloat16)
    lT = lax.dot_general(w2_ref[...].astype(jnp.bfloat16), hT,
                         (((0,), (0,)), ((), ())),
                         preferred_element_type=jnp.float32)
    o_ref[...] = jax.nn.sigmoid(lT + b2_ref[...])[0]


def _prob_table(table, W1, b1, W2, b2, interpret=False):
    grid = pl.cdiv(N_PRED, BLK)
    return pl.pallas_call(
        _mlp_body,
        grid=(grid,),
        in_specs=[
            pl.BlockSpec((EMB_DIM, BLK), lambda i: (0, i)),
            pl.BlockSpec((EMB_DIM, HID), lambda i: (0, 0)),
            pl.BlockSpec((HID, 1), lambda i: (0, 0)),
            pl.BlockSpec((HID, 1), lambda i: (0, 0)),
            pl.BlockSpec((1, 1), lambda i: (0, 0)),
        ],
        out_specs=pl.BlockSpec((BLK,), lambda i: (i,)),
        out_shape=jax.ShapeDtypeStruct((N_PRED,), jnp.float32),
        interpret=interpret,
    )(table.T, W1, b1.reshape(HID, 1), W2, b2.reshape(1, 1))


_NC, _NS = 2, 16
_NW = _NC * _NS
_TOT = B * F
_PER_W = _TOT // _NW


def _gather_body(ptab_hbm, idx_hbm, out_hbm, idx_v, val_v, sem):
    wid = lax.axis_index("s") * _NC + lax.axis_index("c")
    base = wid * _PER_W
    pltpu.sync_copy(idx_hbm.at[pl.ds(base, _PER_W)], idx_v)
    pltpu.async_copy(ptab_hbm.at[idx_v], val_v, sem).wait()
    pltpu.sync_copy(val_v, out_hbm.at[pl.ds(base, _PER_W)])


@functools.cache
def _make_gather():
    return pl.kernel(
        _gather_body,
        out_type=jax.ShapeDtypeStruct((_TOT,), jnp.float32),
        mesh=plsc.VectorSubcoreMesh(
            core_axis_name="c", subcore_axis_name="s",
            num_cores=_NC, num_subcores=_NS),
        scratch_types=[
            pltpu.VMEM((_PER_W,), jnp.int32),
            pltpu.VMEM((_PER_W,), jnp.float32),
            pltpu.SemaphoreType.DMA,
        ],
    )


def kernel(predicate_indices, table, W1, b1, W2, b2):
    ptab = _prob_table(table, W1, b1, W2, b2)
    flat_idx = predicate_indices.T.reshape(_TOT).astype(jnp.int32)
    probs = _make_gather()(ptab, flat_idx)
    return probs.reshape(F, B).T
